# baseline (speedup 1.0000x reference)
import math

import jax
import jax.numpy as jnp
from jax import lax
from jax.experimental import pallas as pl
from jax.experimental.pallas import tpu as pltpu
from jax.experimental.pallas import tpu_sc as plsc

H = 16
B = 32
I = 2048
J = 2048
D = I + J
R = 16
SW = D - R
SCALE_F = 0.125
MAX_DIST = 128
NCHUNK = D // 16
SCHUNK = SW // 16


def _sc_body(q_hbm, tab_hbm, out_hbm, q_v, tab_v, v_v, s_v, sem):
    head = lax.axis_index("s")
    half = lax.axis_index("c")

    pltpu.sync_copy(q_hbm, q_v)
    pltpu.sync_copy(tab_hbm, tab_v)

    lane = lax.iota(jnp.int32, 16)
    head_vec = jnp.full((16,), head, dtype=jnp.int32)

    def bucket_chunk(c, carry):
        d = c * 16 + lane
        n_signed = (I - 1) - d
        ret_base = jnp.where(n_signed < 0, B // 2, 0)
        n_abs = jnp.abs(n_signed)
        is_small = n_abs < (B // 4)
        q = q_v[pl.ds(c * 16, 16)]
        val_large = (B // 4) + q.astype(jnp.int32)
        val_large = jnp.minimum(val_large, (B // 2) - 1)
        bucket = ret_base + jnp.where(is_small, n_abs, val_large)
        rows = plsc.load_gather(tab_v, [bucket * H + head_vec])
        v_v[pl.ds(c * 16, 16)] = rows * SCALE_F
        return carry

    lax.fori_loop(0, NCHUNK, bucket_chunk, 0)

    def shift_chunk(c, carry):
        base = c * 16
        for r in range(R):
            idx = base + (R - 1 - r) + lane
            s_v[r, pl.ds(base, 16)] = plsc.load_gather(v_v, [idx])
        return carry

    lax.fori_loop(0, SCHUNK, shift_chunk, 0)

    def dma_group(g, carry):
        i0 = half * (I // 2) + g * R
        a = pl.multiple_of((I - R) - i0, R)
        pltpu.async_copy(
            s_v.at[:, pl.ds(a, J)],
            out_hbm.at[head, pl.ds(i0, R), :],
            sem,
        ).wait()
        return carry

    lax.fori_loop(0, I // 2 // R, dma_group, 0)


@jax.jit
def _bias_sc(q, table):
    mesh = plsc.VectorSubcoreMesh(
        core_axis_name="c", subcore_axis_name="s", num_cores=2, num_subcores=16
    )
    return pl.kernel(
        _sc_body,
        out_type=jax.ShapeDtypeStruct((H, I, J), jnp.float32),
        mesh=mesh,
        scratch_types=[
            pltpu.VMEM((D,), jnp.float32),
            pltpu.VMEM((B * H,), jnp.float32),
            pltpu.VMEM((D,), jnp.float32),
            pltpu.VMEM((R, SW), jnp.float32),
            pltpu.SemaphoreType.DMA,
        ],
        compiler_params=pltpu.CompilerParams(
            use_tc_tiling_on_sc=False,
            needs_layout_passes=False,
        ),
        name="t5_rel_bias_sc",
    )(q, table)


def kernel(x, relative_attention_bias):
    d = jnp.arange(D, dtype=jnp.int32)
    n_abs = jnp.abs((I - 1) - d)
    t = jnp.log(n_abs.astype(jnp.float32) / (B // 4))
    t = t / math.log(MAX_DIST / (B // 4))
    t = t * ((B // 2) - (B // 4))
    q = jnp.where(n_abs < (B // 4), 0.0, t)

    out = _bias_sc(q, relative_attention_bias.reshape(-1))
    return out.reshape(1, H, 1, I, J)


# speedup vs baseline: 1.0032x; 1.0032x over previous
import math

import jax
import jax.numpy as jnp
from jax import lax
from jax.experimental import pallas as pl
from jax.experimental.pallas import tpu as pltpu
from jax.experimental.pallas import tpu_sc as plsc

H = 16
B = 32
I = 2048
J = 2048
D = I + J
R = 16
SW = D - R
SCALE_F = 0.125
MAX_DIST = 128
NCHUNK = D // 16
SCHUNK = SW // 16


def _sc_body(q_hbm, tab_hbm, out_hbm, q_v, tab_v, v_v, s_v, sem):
    head = lax.axis_index("s")
    half = lax.axis_index("c")

    pltpu.sync_copy(q_hbm, q_v)
    pltpu.sync_copy(tab_hbm, tab_v)

    lane = lax.iota(jnp.int32, 16)
    head_vec = jnp.full((16,), head, dtype=jnp.int32)

    def bucket_chunk(c, carry):
        d = c * 16 + lane
        n_signed = (I - 1) - d
        ret_base = jnp.where(n_signed < 0, B // 2, 0)
        n_abs = jnp.abs(n_signed)
        is_small = n_abs < (B // 4)
        q = q_v[pl.ds(c * 16, 16)]
        val_large = (B // 4) + q.astype(jnp.int32)
        val_large = jnp.minimum(val_large, (B // 2) - 1)
        bucket = ret_base + jnp.where(is_small, n_abs, val_large)
        rows = plsc.load_gather(tab_v, [bucket * H + head_vec])
        v_v[pl.ds(c * 16, 16)] = rows * SCALE_F
        return carry

    lax.fori_loop(0, NCHUNK, bucket_chunk, 0)

    def shift_chunk(c, carry):
        base = c * 16
        for r in range(R):
            idx = base + (R - 1 - r) + lane
            s_v[r, pl.ds(base, 16)] = plsc.load_gather(v_v, [idx])
        return carry

    lax.fori_loop(0, SCHUNK, shift_chunk, 0)

    def dma_group(g, carry):
        i0 = half * (I // 2) + g * R
        a = pl.multiple_of((I - R) - i0, R)
        pltpu.async_copy(
            s_v.at[:, pl.ds(a, J)],
            out_hbm.at[0, head, 0, pl.ds(i0, R), :],
            sem,
        ).wait()
        return carry

    lax.fori_loop(0, I // 2 // R, dma_group, 0)


@jax.jit
def _bias_sc(q, table):
    mesh = plsc.VectorSubcoreMesh(
        core_axis_name="c", subcore_axis_name="s", num_cores=2, num_subcores=16
    )
    return pl.kernel(
        _sc_body,
        out_type=jax.ShapeDtypeStruct((1, H, 1, I, J), jnp.float32),
        mesh=mesh,
        scratch_types=[
            pltpu.VMEM((D,), jnp.float32),
            pltpu.VMEM((B * H,), jnp.float32),
            pltpu.VMEM((D,), jnp.float32),
            pltpu.VMEM((R, SW), jnp.float32),
            pltpu.SemaphoreType.DMA,
        ],
        compiler_params=pltpu.CompilerParams(
            use_tc_tiling_on_sc=False,
            needs_layout_passes=False,
        ),
        name="t5_rel_bias_sc",
    )(q, table)


def kernel(x, relative_attention_bias):
    d = jnp.arange(D, dtype=jnp.int32)
    n_abs = jnp.abs((I - 1) - d)
    t = jnp.log(n_abs.astype(jnp.float32) / (B // 4))
    t = t / math.log(MAX_DIST / (B // 4))
    t = t * ((B // 2) - (B // 4))
    q = jnp.where(n_abs < (B // 4), 0.0, t)

    return _bias_sc(q, relative_attention_bias.reshape(-1))


# speedup vs baseline: 1.2354x; 1.2315x over previous
import math

import jax
import jax.numpy as jnp
from jax import lax
from jax.experimental import pallas as pl
from jax.experimental.pallas import tpu as pltpu
from jax.experimental.pallas import tpu_sc as plsc

H = 16
B = 32
I = 2048
J = 2048
D = I + J
TI = I // 8
TJ = J // 128
TK = 31
SCALE_F = 0.125
MAX_DIST = 128
NCHUNK = D // 16


def _sc_body(q_hbm, tab_hbm, out_hbm, q_v, tab_v, v_v, st0_v, st1_v, sem):
    s = lax.axis_index("s")
    c = lax.axis_index("c")

    pltpu.sync_copy(q_hbm, q_v)
    pltpu.sync_copy(tab_hbm, tab_v)

    lane = lax.iota(jnp.int32, 16)
    shift_c = 127 - 8 * s

    st_bufs = (st0_v, st1_v)
    handles = {}

    for h in range(H):
        for cp in handles.pop(h - 2, ()):
            cp.wait()

        def bucket_chunk(ch, carry):
            d = ch * 16 + lane
            n_signed = (I - 1) - d
            ret_base = jnp.where(n_signed < 0, B // 2, 0)
            n_abs = jnp.abs(n_signed)
            is_small = n_abs < (B // 4)
            q = q_v[pl.ds(ch * 16, 16)]
            val_large = (B // 4) + q.astype(jnp.int32)
            val_large = jnp.minimum(val_large, (B // 2) - 1)
            bucket = ret_base + jnp.where(is_small, n_abs, val_large)
            v_v[pl.ds(ch * 16, 16)] = (
                plsc.load_gather(tab_v, [bucket * H + h]) * SCALE_F
            )
            return carry

        lax.fori_loop(0, NCHUNK, bucket_chunk, 0)

        st_v = st_bufs[h % 2]

        def st_chunk(ch, carry):
            fl = ch * 16
            tk = fl // 1024
            rem = fl - tk * 1024
            si = rem // 128
            sjb = rem - si * 128
            idx = tk * 128 + shift_c + sjb - si + lane
            st_v[tk, si, pl.ds(sjb, 16)] = plsc.load_gather(v_v, [idx])
            return carry

        lax.fori_loop(0, TK * 64, st_chunk, 0)

        cps = []
        for mm in range(8):
            ti = 128 * c + 16 * mm + s
            tk0 = 15 - 8 * c - mm
            cps.append(
                pltpu.async_copy(
                    st_v.at[pl.ds(tk0, TJ), :, :],
                    out_hbm.at[h, ti, :, :, :],
                    sem,
                )
            )
        handles[h] = cps

    for hh in (H - 2, H - 1):
        for cp in handles.pop(hh, ()):
            cp.wait()


@jax.jit
def _bias_sc(q, table):
    mesh = plsc.VectorSubcoreMesh(
        core_axis_name="c", subcore_axis_name="s", num_cores=2, num_subcores=16
    )
    return pl.kernel(
        _sc_body,
        out_type=jax.ShapeDtypeStruct((H, TI, TJ, 8, 128), jnp.float32),
        mesh=mesh,
        scratch_types=[
            pltpu.VMEM((D,), jnp.float32),
            pltpu.VMEM((B * H,), jnp.float32),
            pltpu.VMEM((D,), jnp.float32),
            pltpu.VMEM((TK, 8, 128), jnp.float32),
            pltpu.VMEM((TK, 8, 128), jnp.float32),
            pltpu.SemaphoreType.DMA,
        ],
        compiler_params=pltpu.CompilerParams(
            use_tc_tiling_on_sc=False,
            needs_layout_passes=False,
        ),
        name="t5_rel_bias_sc",
    )(q, table)


def kernel(x, relative_attention_bias):
    d = jnp.arange(D, dtype=jnp.int32)
    n_abs = jnp.abs((I - 1) - d)
    t = jnp.log(n_abs.astype(jnp.float32) / (B // 4))
    t = t / math.log(MAX_DIST / (B // 4))
    t = t * ((B // 2) - (B // 4))
    q = jnp.where(n_abs < (B // 4), 0.0, t)

    out_t = _bias_sc(q, relative_attention_bias.reshape(-1))
    return out_t.transpose(0, 1, 3, 2, 4).reshape(1, H, 1, I, J)


# speedup vs baseline: 3.4848x; 2.8208x over previous
import math

import jax
import jax.numpy as jnp
from jax import lax
from jax.experimental import pallas as pl
from jax.experimental.pallas import tpu as pltpu
from jax.experimental.pallas import tpu_sc as plsc

H = 16
B = 32
I = 2048
J = 2048
D = I + J
TI = I // 8
TJ = J // 128
TK = 23
SCALE_F = 0.125
MAX_DIST = 128
NCHUNK = D // 16


def _sc_body(q_hbm, tab_hbm, out_hbm, q_v, tab_v, v_v, st0_v, st1_v, sem):
    s = lax.axis_index("s")
    c = lax.axis_index("c")

    pltpu.sync_copy(q_hbm, q_v)
    pltpu.sync_copy(tab_hbm, tab_v)

    lane = lax.iota(jnp.int32, 16)
    shift_c = 127 - 8 * s
    tkoff = 8 * (1 - c)

    st_bufs = (st0_v, st1_v)
    handles = {}

    for h in range(H):
        for cp in handles.pop(h - 2, ()):
            cp.wait()

        @plsc.parallel_loop(0, NCHUNK, 1, unroll=4)
        def bucket_chunk(ch):
            d = ch * 16 + lane
            n_signed = (I - 1) - d
            ret_base = jnp.where(n_signed < 0, B // 2, 0)
            n_abs = jnp.abs(n_signed)
            is_small = n_abs < (B // 4)
            q = q_v[pl.ds(ch * 16, 16)]
            val_large = (B // 4) + q.astype(jnp.int32)
            val_large = jnp.minimum(val_large, (B // 2) - 1)
            bucket = ret_base + jnp.where(is_small, n_abs, val_large)
            v_v[pl.ds(ch * 16, 16)] = (
                plsc.load_gather(tab_v, [bucket * H + h]) * SCALE_F
            )

        st_v = st_bufs[h % 2]

        @plsc.parallel_loop(0, TK * 8, 1, unroll=2)
        def st_row(it):
            tk = it // 8
            si = it - 8 * tk
            base = (tk + tkoff) * 128 + shift_c - si
            for sjc in range(0, 128, 16):
                st_v[tk, si, pl.ds(sjc, 16)] = v_v[pl.ds(base + sjc, 16)]

        cps = []
        for mm in range(8):
            ti = 128 * c + 16 * mm + s
            tk0 = 7 - mm
            cps.append(
                pltpu.async_copy(
                    st_v.at[pl.ds(tk0, TJ), :, :],
                    out_hbm.at[h, ti, :, :, :],
                    sem,
                )
            )
        handles[h] = cps

    for hh in (H - 2, H - 1):
        for cp in handles.pop(hh, ()):
            cp.wait()


@jax.jit
def _bias_sc(q, table):
    mesh = plsc.VectorSubcoreMesh(
        core_axis_name="c", subcore_axis_name="s", num_cores=2, num_subcores=16
    )
    return pl.kernel(
        _sc_body,
        out_type=jax.ShapeDtypeStruct((H, TI, TJ, 8, 128), jnp.float32),
        mesh=mesh,
        scratch_types=[
            pltpu.VMEM((D,), jnp.float32),
            pltpu.VMEM((B * H,), jnp.float32),
            pltpu.VMEM((D,), jnp.float32),
            pltpu.VMEM((TK, 8, 128), jnp.float32),
            pltpu.VMEM((TK, 8, 128), jnp.float32),
            pltpu.SemaphoreType.DMA,
        ],
        compiler_params=pltpu.CompilerParams(
            use_tc_tiling_on_sc=False,
            needs_layout_passes=False,
        ),
        name="t5_rel_bias_sc",
    )(q, table)


def kernel(x, relative_attention_bias):
    d = jnp.arange(D, dtype=jnp.int32)
    n_abs = jnp.abs((I - 1) - d)
    t = jnp.log(n_abs.astype(jnp.float32) / (B // 4))
    t = t / math.log(MAX_DIST / (B // 4))
    t = t * ((B // 2) - (B // 4))
    q = jnp.where(n_abs < (B // 4), 0.0, t)

    out_t = _bias_sc(q, relative_attention_bias.reshape(-1))
    return out_t.transpose(0, 1, 3, 2, 4).reshape(1, H, 1, I, J)
